# baseline (speedup 1.0000x reference)
import functools

import jax
import jax.numpy as jnp
from jax import lax
from jax.experimental import pallas as pl
from jax.experimental.pallas import tpu as pltpu
from jax.experimental.pallas import tpu_sc as plsc

_NS_IN = 2562
_NS_OUT = 10242
_NALPHA = 6
_B = 8
_C = 128
_R = _B * _C * _NALPHA
_NBLK = (_NS_OUT + 15) // 16
_NPAD = _NBLK * 16

_NC = 2
_NS = 16
_NW = _NC * _NS
_ROWS_PER_W = _R // _NW


def _make_unpool():
    mesh = plsc.VectorSubcoreMesh(core_axis_name="c", subcore_axis_name="s")

    @functools.partial(
        pl.kernel,
        mesh=mesh,
        compiler_params=pltpu.CompilerParams(
            use_tc_tiling_on_sc=False, needs_layout_passes=False
        ),
        out_type=jax.ShapeDtypeStruct((_R, _NS_OUT), jnp.float32),
        scratch_types=[
            pltpu.VMEM((_NPAD,), jnp.int32),
            pltpu.VMEM((_NPAD,), jnp.int32),
            pltpu.VMEM((_NS_IN,), jnp.float32),
            pltpu.VMEM((_NPAD,), jnp.float32),
        ],
    )
    def unpool(x_hbm, i0_hbm, i1_hbm, out_hbm, i0_v, i1_v, row_v, out_v):
        wid = lax.axis_index("s") * _NC + lax.axis_index("c")
        pltpu.sync_copy(i0_hbm, i0_v)
        pltpu.sync_copy(i1_hbm, i1_v)

        def row_body(t, carry):
            r = wid * _ROWS_PER_W + t
            pltpu.sync_copy(x_hbm.at[r], row_v)

            def blk_body(j, carry2):
                o = pl.multiple_of(j * 16, 16)
                i0 = i0_v[pl.ds(o, 16)]
                i1 = i1_v[pl.ds(o, 16)]
                g0 = plsc.load_gather(row_v, [i0])
                g1 = plsc.load_gather(row_v, [i1])
                out_v[pl.ds(o, 16)] = (g0 + g1) * 0.5
                return carry2

            lax.fori_loop(0, _NBLK, blk_body, 0)
            pltpu.sync_copy(out_v.at[pl.ds(0, _NS_OUT)], out_hbm.at[r])
            return carry

        lax.fori_loop(0, _ROWS_PER_W, row_body, 0)

    return unpool


_unpool = _make_unpool()


def kernel(x, index):
    idx = index.astype(jnp.int32)
    i0 = jnp.pad(idx[:_NS_OUT, 0] % _NS_IN, (0, _NPAD - _NS_OUT))
    i1 = jnp.pad(idx[:_NS_OUT, 1] % _NS_IN, (0, _NPAD - _NS_OUT))
    xr = x.reshape(_R, _NS_IN)
    out = _unpool(xr, i0, i1)
    return out.reshape(_B, _C, _NALPHA * _NS_OUT)


# speedup vs baseline: 1.0618x; 1.0618x over previous
import functools

import jax
import jax.numpy as jnp
from jax import lax
from jax.experimental import pallas as pl
from jax.experimental.pallas import tpu as pltpu
from jax.experimental.pallas import tpu_sc as plsc

_NS_IN = 2562
_NS_OUT = 10242
_NALPHA = 6
_B = 8
_C = 128
_R = _B * _C * _NALPHA
_NBLK = 641
_NPAD = _NBLK * 16

_NC = 2
_NS = 16
_NW = _NC * _NS
_ROWS_PER_W = _R // _NW
_T = 4
_NGRP = _ROWS_PER_W // _T


def _make_unpool():
    mesh = plsc.VectorSubcoreMesh(core_axis_name="c", subcore_axis_name="s")

    @functools.partial(
        pl.kernel,
        mesh=mesh,
        compiler_params=pltpu.CompilerParams(
            use_tc_tiling_on_sc=False, needs_layout_passes=False
        ),
        out_type=jax.ShapeDtypeStruct((_R, _NS_OUT), jnp.float32),
        scratch_types=(
            [pltpu.VMEM((_NPAD,), jnp.int32)] * 2
            + [pltpu.VMEM((_NS_IN,), jnp.float32)] * (2 * _T)
            + [pltpu.VMEM((_NPAD,), jnp.float32)] * (2 * _T)
            + [pltpu.SemaphoreType.DMA] * 4
        ),
    )
    def unpool(x_hbm, i0_hbm, i1_hbm, out_hbm, *sc):
        i0_v, i1_v = sc[0], sc[1]
        rows_v = (sc[2:2 + _T], sc[2 + _T:2 + 2 * _T])
        outs_v = (sc[2 + 2 * _T:2 + 3 * _T], sc[2 + 3 * _T:2 + 4 * _T])
        sems_r = (sc[-4], sc[-3])
        sems_o = (sc[-2], sc[-1])

        wid = lax.axis_index("s") * _NC + lax.axis_index("c")
        base_row = wid * _ROWS_PER_W

        pltpu.sync_copy(i0_hbm, i0_v)
        pltpu.sync_copy(i1_hbm, i1_v)

        def start_in(s, r):
            for t in range(_T):
                pltpu.async_copy(x_hbm.at[r + t], rows_v[s][t], sems_r[s])

        def wait_in(s, r):
            for t in range(_T):
                pltpu.make_async_copy(
                    x_hbm.at[r + t], rows_v[s][t], sems_r[s]
                ).wait()

        def start_out(s, r):
            for t in range(_T):
                pltpu.async_copy(
                    outs_v[s][t].at[pl.ds(0, _NS_OUT)],
                    out_hbm.at[r + t],
                    sems_o[s],
                )

        def wait_out(s, r):
            for t in range(_T):
                pltpu.make_async_copy(
                    outs_v[s][t].at[pl.ds(0, _NS_OUT)],
                    out_hbm.at[r + t],
                    sems_o[s],
                ).wait()

        def compute_group(s):
            def jblk(j, c2):
                o = pl.multiple_of(j * 16, 16)
                i0 = i0_v[pl.ds(o, 16)]
                i1 = i1_v[pl.ds(o, 16)]
                for t in range(_T):
                    g0 = plsc.load_gather(rows_v[s][t], [i0])
                    g1 = plsc.load_gather(rows_v[s][t], [i1])
                    outs_v[s][t][pl.ds(o, 16)] = (g0 + g1) * 0.5
                return c2

            lax.fori_loop(0, _NBLK, jblk, 0, unroll=4)

        def iter_group(s, r, drain, prefetch):
            wait_in(s, r)
            if drain:
                wait_out(s, r)
            compute_group(s)
            start_out(s, r)
            if prefetch:
                start_in(s, r + 2 * _T)

        for s in (0, 1):
            start_in(s, base_row + s * _T)
        for s in (0, 1):
            iter_group(s, base_row + s * _T, drain=False, prefetch=True)

        def outer(h, carry):
            for s in (0, 1):
                r = base_row + (2 + 2 * h + s) * _T
                iter_group(s, r, drain=True, prefetch=True)
            return carry

        lax.fori_loop(0, (_NGRP - 4) // 2, outer, 0)

        for s in (0, 1):
            iter_group(s, base_row + (_NGRP - 2 + s) * _T,
                       drain=True, prefetch=False)
        for s in (0, 1):
            wait_out(s, base_row + (_NGRP - 2 + s) * _T)

    return unpool


_unpool = _make_unpool()


def kernel(x, index):
    idx = index.astype(jnp.int32)
    i0 = jnp.pad(idx[:_NS_OUT, 0] % _NS_IN, (0, _NPAD - _NS_OUT))
    i1 = jnp.pad(idx[:_NS_OUT, 1] % _NS_IN, (0, _NPAD - _NS_OUT))
    xr = x.reshape(_R, _NS_IN)
    out = _unpool(xr, i0, i1)
    return out.reshape(_B, _C, _NALPHA * _NS_OUT)


# speedup vs baseline: 2.4207x; 2.2798x over previous
import functools

import jax
import jax.numpy as jnp
from jax import lax
from jax.experimental import pallas as pl
from jax.experimental.pallas import tpu as pltpu
from jax.experimental.pallas import tpu_sc as plsc

_NS_IN = 2562
_NS_OUT = 10242
_NALPHA = 6
_B = 8
_C = 128
_NBLK = 641
_NPAD = _NBLK * 16
_XW = _NALPHA * _NS_IN
_OW = _NALPHA * _NS_OUT
_SPLIT = 4 * _NS_OUT

_NC = 2
_NS = 16
_NW = _NC * _NS
_P = _B * _C
_PPW = _P // _NW


def _make_unpool():
    mesh = plsc.VectorSubcoreMesh(core_axis_name="c", subcore_axis_name="s")

    @functools.partial(
        pl.kernel,
        mesh=mesh,
        compiler_params=pltpu.CompilerParams(
            use_tc_tiling_on_sc=False, needs_layout_passes=False
        ),
        out_type=jax.ShapeDtypeStruct((_P, _OW), jnp.float32),
        scratch_types=[
            pltpu.VMEM((_NPAD,), jnp.int32),
            pltpu.VMEM((_NPAD,), jnp.int32),
            pltpu.VMEM((_XW,), jnp.float32),
            pltpu.VMEM((_XW,), jnp.float32),
            pltpu.VMEM((_OW,), jnp.float32),
            pltpu.SemaphoreType.DMA,
            pltpu.SemaphoreType.DMA,
            pltpu.SemaphoreType.DMA,
            pltpu.SemaphoreType.DMA,
        ],
    )
    def unpool(x_hbm, i0_hbm, i1_hbm, out_hbm,
               i0_v, i1_v, in0_v, in1_v, out_v, si0, si1, so1, so2):
        ins_v = (in0_v, in1_v)
        sems_i = (si0, si1)

        wid = lax.axis_index("s") * _NC + lax.axis_index("c")
        p0 = wid * _PPW

        pltpu.sync_copy(i0_hbm, i0_v)
        pltpu.sync_copy(i1_hbm, i1_v)

        def piece1(p):
            return pltpu.make_async_copy(
                out_v.at[pl.ds(0, _SPLIT)],
                out_hbm.at[p, pl.ds(0, _SPLIT)],
                so1,
            )

        def piece2(p):
            return pltpu.make_async_copy(
                out_v.at[pl.ds(_SPLIT, _OW - _SPLIT)],
                out_hbm.at[p, pl.ds(_SPLIT, _OW - _SPLIT)],
                so2,
            )

        def in_copy(s, p):
            return pltpu.make_async_copy(x_hbm.at[p], ins_v[s], sems_i[s])

        def gather_pass(s, alphas):
            @plsc.parallel_loop(0, _NBLK, unroll=4)
            def jblk(j):
                o = pl.multiple_of(j * 16, 16)
                i0 = i0_v[pl.ds(o, 16)]
                i1 = i1_v[pl.ds(o, 16)]
                ovec = lax.broadcasted_iota(jnp.int32, (16,), 0) + o
                m = ovec < _NS_OUT
                for a in alphas:
                    g0 = plsc.load_gather(ins_v[s], [i0 + a * _NS_IN])
                    g1 = plsc.load_gather(ins_v[s], [i1 + a * _NS_IN])
                    plsc.store_scatter(
                        out_v, [ovec + a * _NS_OUT], (g0 + g1) * 0.5, mask=m
                    )

        def iter_unit(s, p, first, prefetch):
            in_copy(s, p).wait()
            if not first:
                piece1(p).wait()
            gather_pass(s, (0, 1, 2, 3))
            piece1(p).start()
            if not first:
                piece2(p).wait()
            gather_pass(s, (4, 5))
            piece2(p).start()
            if prefetch:
                in_copy(s, p + 2).start()

        in_copy(0, p0).start()
        in_copy(1, p0 + 1).start()

        iter_unit(0, p0, first=True, prefetch=True)

        def outer(h, carry):
            k = 2 * h + 1
            iter_unit(1, p0 + k, first=False, prefetch=True)
            iter_unit(0, p0 + k + 1, first=False, prefetch=True)
            return carry

        lax.fori_loop(0, (_PPW - 4) // 2, outer, 0)

        iter_unit(1, p0 + _PPW - 3, first=False, prefetch=True)
        iter_unit(0, p0 + _PPW - 2, first=False, prefetch=False)
        iter_unit(1, p0 + _PPW - 1, first=False, prefetch=False)
        piece1(p0 + _PPW - 1).wait()
        piece2(p0 + _PPW - 1).wait()

    return unpool


_unpool = _make_unpool()


def kernel(x, index):
    idx = index.astype(jnp.int32)
    i0 = jnp.pad(idx[:_NS_OUT, 0] % _NS_IN, (0, _NPAD - _NS_OUT))
    i1 = jnp.pad(idx[:_NS_OUT, 1] % _NS_IN, (0, _NPAD - _NS_OUT))
    xr = x.reshape(_P, _XW)
    out = _unpool(xr, i0, i1)
    return out.reshape(_B, _C, _OW)


# speedup vs baseline: 2.4240x; 1.0014x over previous
import functools

import jax
import jax.numpy as jnp
from jax import lax
from jax.experimental import pallas as pl
from jax.experimental.pallas import tpu as pltpu
from jax.experimental.pallas import tpu_sc as plsc

_NS_IN = 2562
_NS_OUT = 10242
_NALPHA = 6
_B = 8
_C = 128
_NBLK = 641
_NPAD = _NBLK * 16
_XW = _NALPHA * _NS_IN
_OW = _NALPHA * _NS_OUT
_SPLIT = 4 * _NS_OUT

_NC = 2
_NS = 16
_NW = _NC * _NS
_P = _B * _C
_PPW = _P // _NW


def _make_unpool():
    mesh = plsc.VectorSubcoreMesh(core_axis_name="c", subcore_axis_name="s")

    @functools.partial(
        pl.kernel,
        mesh=mesh,
        compiler_params=pltpu.CompilerParams(
            use_tc_tiling_on_sc=False, needs_layout_passes=False
        ),
        out_type=jax.ShapeDtypeStruct((_B, _C, _OW), jnp.float32),
        scratch_types=[
            pltpu.VMEM((_NPAD,), jnp.int32),
            pltpu.VMEM((_NPAD,), jnp.int32),
            pltpu.VMEM((_XW,), jnp.float32),
            pltpu.VMEM((_XW,), jnp.float32),
            pltpu.VMEM((_OW,), jnp.float32),
            pltpu.SemaphoreType.DMA,
            pltpu.SemaphoreType.DMA,
            pltpu.SemaphoreType.DMA,
            pltpu.SemaphoreType.DMA,
        ],
    )
    def unpool(x_hbm, i0_hbm, i1_hbm, out_hbm,
               i0_v, i1_v, in0_v, in1_v, out_v, si0, si1, so1, so2):
        ins_v = (in0_v, in1_v)
        sems_i = (si0, si1)

        wid = lax.axis_index("s") * _NC + lax.axis_index("c")
        b = wid // 4
        c0 = (wid % 4) * _PPW

        pltpu.sync_copy(i0_hbm, i0_v)
        pltpu.sync_copy(i1_hbm, i1_v)

        def piece1(c):
            return pltpu.make_async_copy(
                out_v.at[pl.ds(0, _SPLIT)],
                out_hbm.at[b, c, pl.ds(0, _SPLIT)],
                so1,
            )

        def piece2(c):
            return pltpu.make_async_copy(
                out_v.at[pl.ds(_SPLIT, _OW - _SPLIT)],
                out_hbm.at[b, c, pl.ds(_SPLIT, _OW - _SPLIT)],
                so2,
            )

        def in_copy(s, c):
            return pltpu.make_async_copy(x_hbm.at[b, c], ins_v[s], sems_i[s])

        def gather_pass(s, alphas):
            @plsc.parallel_loop(0, _NBLK, unroll=4)
            def jblk(j):
                o = pl.multiple_of(j * 16, 16)
                i0 = i0_v[pl.ds(o, 16)]
                i1 = i1_v[pl.ds(o, 16)]
                ovec = lax.broadcasted_iota(jnp.int32, (16,), 0) + o
                m = ovec < _NS_OUT
                for a in alphas:
                    g0 = plsc.load_gather(ins_v[s], [i0 + a * _NS_IN])
                    g1 = plsc.load_gather(ins_v[s], [i1 + a * _NS_IN])
                    plsc.store_scatter(
                        out_v, [ovec + a * _NS_OUT], (g0 + g1) * 0.5, mask=m
                    )

        def iter_unit(s, c, first, prefetch):
            in_copy(s, c).wait()
            if not first:
                piece1(c).wait()
            gather_pass(s, (0, 1, 2, 3))
            piece1(c).start()
            if not first:
                piece2(c).wait()
            gather_pass(s, (4, 5))
            piece2(c).start()
            if prefetch:
                in_copy(s, c + 2).start()

        in_copy(0, c0).start()
        in_copy(1, c0 + 1).start()

        iter_unit(0, c0, first=True, prefetch=True)

        def outer(h, carry):
            k = 2 * h + 1
            iter_unit(1, c0 + k, first=False, prefetch=True)
            iter_unit(0, c0 + k + 1, first=False, prefetch=True)
            return carry

        lax.fori_loop(0, (_PPW - 4) // 2, outer, 0)

        iter_unit(1, c0 + _PPW - 3, first=False, prefetch=True)
        iter_unit(0, c0 + _PPW - 2, first=False, prefetch=False)
        iter_unit(1, c0 + _PPW - 1, first=False, prefetch=False)
        piece1(c0 + _PPW - 1).wait()
        piece2(c0 + _PPW - 1).wait()

    return unpool


_unpool = _make_unpool()


def kernel(x, index):
    idx = index.astype(jnp.int32)
    i0 = jnp.pad(idx[:_NS_OUT, 0] % _NS_IN, (0, _NPAD - _NS_OUT))
    i1 = jnp.pad(idx[:_NS_OUT, 1] % _NS_IN, (0, _NPAD - _NS_OUT))
    return _unpool(x, i0, i1)
